# Initial kernel scaffold; baseline (speedup 1.0000x reference)
import jax
import jax.numpy as jnp
from jax.experimental import pallas as pl


def kernel(X, random_rotations, W_embed, b_embed, W_enc_id, b_enc_id, W_t_id, b_t_id, W_h_id, theta_id, W_dec_id, b_dec_id, W_out_id, b_out_id, W_enc_reg, b_enc_reg, W_t_reg, b_t_reg, W_h_reg, theta_reg, W_dec_reg, b_dec_reg, W_out_reg, b_out_reg):
    raise NotImplementedError("write your pallas kernel here")


# speedup vs baseline: 22.4097x; 22.4097x over previous
import functools

import jax
import jax.numpy as jnp
from jax import lax
from jax.experimental import pallas as pl
from jax.experimental.pallas import tpu as pltpu
from jax.experimental.pallas import tpu_sc as plsc

B = 10
N = 5000
NFEAT = 15
NCLS = 12
ENCF = NCLS + NFEAT - 1
DDIM = 256
HID = 256
NBINS = 10
K = 5
NID = 8
NREG = 4
CHUNK = N // NBINS
NCHUNKS = B * NBINS
ROWS = B * N
SC_CH = 112
NW = 32
ROWS_PAD = ((ROWS + NW * SC_CH - 1) // (NW * SC_CH)) * (NW * SC_CH)
PER_W = ROWS_PAD // NW
NCH = PER_W // SC_CH
RBLK = 1000


def _encode(x):
    m = x.shape[0]
    cls = x[:, 0:1].astype(jnp.int32)
    iota = lax.broadcasted_iota(jnp.int32, (m, NCLS), 1)
    oh = (cls == iota).astype(jnp.float32)
    return jnp.concatenate([oh, x[:, 1:NFEAT]], axis=1)


def _selu(x):
    alpha = 1.6732632423543772848170429916717
    scale = 1.0507009873554804934193349852946
    return scale * jnp.where(x > 0, x, alpha * (jnp.exp(x) - 1.0))


def _rank_kernel(x_ref, wemb_ref, bemb_ref, rot_ref, out_ref):
    b = pl.program_id(0)
    x = x_ref[0]
    enc = _encode(x)
    pts = _selu(jnp.dot(enc, wemb_ref[...], preferred_element_type=jnp.float32)
                + bemb_ref[...])
    mul = jnp.dot(pts, rot_ref[...], preferred_element_type=jnp.float32)
    full = jnp.concatenate([mul, -mul], axis=1)
    mx = jnp.max(full, axis=1, keepdims=True)
    col10 = lax.broadcasted_iota(jnp.int32, (N, NBINS), 1)
    first = jnp.min(jnp.where(full == mx, col10, 127), axis=1, keepdims=True)
    col16 = lax.broadcasted_iota(jnp.int32, (N, 16), 1)
    ohbin = (col16 == first).astype(jnp.float32)

    tot = jnp.sum(ohbin, axis=0, keepdims=True)
    ltmask = (col16 < first).astype(jnp.float32)
    off_node = jnp.sum(ltmask * tot, axis=1, keepdims=True)

    rl = lax.broadcasted_iota(jnp.int32, (RBLK, RBLK), 0)
    cl = lax.broadcasted_iota(jnp.int32, (RBLK, RBLK), 1)
    ltri = (rl > cl).astype(jnp.float32)

    blockoff = jnp.zeros((1, 16), jnp.float32)
    ranks = []
    for blk in range(N // RBLK):
        ohb = ohbin[blk * RBLK:(blk + 1) * RBLK]
        prefix = jnp.dot(ltri, ohb, preferred_element_type=jnp.float32)
        rv = blockoff + prefix
        within = jnp.sum(ohb * rv, axis=1, keepdims=True)
        ranks.append(within + off_node[blk * RBLK:(blk + 1) * RBLK])
        blockoff = blockoff + jnp.sum(ohb, axis=0, keepdims=True)
    rank = jnp.concatenate(ranks, axis=0)
    out_ref[0] = rank.astype(jnp.int32) + b * N


def _compute_rank(xp, w_embed, b_embed, rot):
    return pl.pallas_call(
        _rank_kernel,
        grid=(B,),
        in_specs=[
            pl.BlockSpec((1, N, 16), lambda i: (i, 0, 0)),
            pl.BlockSpec((ENCF, DDIM), lambda i: (0, 0)),
            pl.BlockSpec((1, DDIM), lambda i: (0, 0)),
            pl.BlockSpec((DDIM, NBINS // 2), lambda i: (0, 0)),
        ],
        out_specs=pl.BlockSpec((1, N, 1), lambda i: (i, 0, 0)),
        out_shape=jax.ShapeDtypeStruct((B, N, 1), jnp.int32),
    )(xp, w_embed, b_embed, rot)


@functools.lru_cache(maxsize=1)
def _sc_kernels():
    mesh = plsc.VectorSubcoreMesh(core_axis_name="c", subcore_axis_name="s")
    scratch = [
        pltpu.VMEM((SC_CH,), jnp.int32),
        pltpu.VMEM((SC_CH, 16), jnp.float32),
        pltpu.SemaphoreType.DMA,
    ]
    out_t = jax.ShapeDtypeStruct((ROWS_PAD, 16), jnp.float32)
    cparams = pltpu.CompilerParams(use_tc_tiling_on_sc=False)

    @functools.partial(pl.kernel, out_type=out_t, mesh=mesh, scratch_types=scratch,
                       compiler_params=cparams)
    def sc_scatter(x_hbm, idx_hbm, out_hbm, idx_v, rows_v, sem):
        wid = lax.axis_index("s") * 2 + lax.axis_index("c")
        base = wid * PER_W
        for j in range(NCH):
            off = base + j * SC_CH
            pltpu.sync_copy(idx_hbm.at[pl.ds(off, SC_CH)], idx_v)
            pltpu.sync_copy(x_hbm.at[pl.ds(off, SC_CH)], rows_v)
            pltpu.async_copy(rows_v, out_hbm.at[idx_v], sem).wait()

    @functools.partial(pl.kernel, out_type=out_t, mesh=mesh, scratch_types=scratch,
                       compiler_params=cparams)
    def sc_gather(y_hbm, idx_hbm, out_hbm, idx_v, rows_v, sem):
        wid = lax.axis_index("s") * 2 + lax.axis_index("c")
        base = wid * PER_W
        for j in range(NCH):
            off = base + j * SC_CH
            pltpu.sync_copy(idx_hbm.at[pl.ds(off, SC_CH)], idx_v)
            pltpu.async_copy(y_hbm.at[idx_v], rows_v, sem).wait()
            pltpu.sync_copy(rows_v, out_hbm.at[pl.ds(off, SC_CH)])

    return sc_scatter, sc_gather


def _main_kernel(xs_ref, wemb_ref, bemb_ref,
                 wencI_ref, bencI_ref, wtI_ref, btI_ref, whI_ref, thI_ref,
                 wdecI_ref, bdecI_ref, woutI_ref, boutI_ref,
                 wencR_ref, bencR_ref, wtR_ref, btR_ref, whR_ref, thR_ref,
                 wdecR_ref, bdecR_ref, woutR_ref, boutR_ref,
                 out_ref):
    xs = xs_ref[0]
    enc = _encode(xs)
    pts = _selu(jnp.dot(enc, wemb_ref[...], preferred_element_type=jnp.float32)
                + bemb_ref[...])
    nrm = pts * lax.rsqrt(jnp.maximum(jnp.sum(pts * pts, axis=1, keepdims=True),
                                      1e-12))
    logits = lax.dot_general(nrm, nrm, (((1,), (1,)), ((), ())),
                             preferred_element_type=jnp.float32)
    lm = jnp.max(logits, axis=1, keepdims=True)
    ex = jnp.exp(logits - lm)
    dm = ex / jnp.sum(ex, axis=1, keepdims=True)

    col = lax.broadcasted_iota(jnp.int32, (CHUNK, CHUNK), 1)
    cur = dm
    sel = jnp.zeros((CHUNK, CHUNK), jnp.bool_)
    for _ in range(K):
        m = jnp.max(cur, axis=1, keepdims=True)
        ismax = cur == m
        fc = jnp.min(jnp.where(ismax, col, 1 << 30), axis=1, keepdims=True)
        s = col == fc
        sel = jnp.logical_or(sel, s)
        cur = jnp.where(s, -1.0, cur)
    a_mat = jnp.where(sel, dm, 0.0)
    indeg = jnp.sum(a_mat, axis=1, keepdims=True)
    norm = lax.rsqrt(indeg + 1e-6)

    def branch(wenc, benc, wt, bt, wh, th, wdec, bdec, wout, bout):
        x = _selu(jnp.dot(enc, wenc, preferred_element_type=jnp.float32) + benc)
        fh = jnp.dot(x, th, preferred_element_type=jnp.float32) * norm
        fh = jnp.dot(a_mat, fh, preferred_element_type=jnp.float32) * norm
        gate = jax.nn.sigmoid(jnp.dot(x, wt, preferred_element_type=jnp.float32) + bt)
        het = jnp.dot(x, wh, preferred_element_type=jnp.float32)
        h = _selu(gate * fh + (1.0 - gate) * het)
        h = _selu(jnp.dot(h, wdec, preferred_element_type=jnp.float32) + bdec)
        return jnp.dot(h, wout, preferred_element_type=jnp.float32) + bout

    y_id = branch(wencI_ref[...], bencI_ref[...], wtI_ref[...], btI_ref[...],
                  whI_ref[...], thI_ref[...], wdecI_ref[...], bdecI_ref[...],
                  woutI_ref[...], boutI_ref[...])
    y_rg = branch(wencR_ref[...], bencR_ref[...], wtR_ref[...], btR_ref[...],
                  whR_ref[...], thR_ref[...], wdecR_ref[...], bdecR_ref[...],
                  woutR_ref[...], boutR_ref[...])
    pad = jnp.zeros((CHUNK, 16 - NID - NREG), jnp.float32)
    out_ref[0] = jnp.concatenate([y_id, y_rg, pad], axis=1)


def _wspec(shape):
    nd = len(shape)
    return pl.BlockSpec(shape, lambda i: (0,) * nd)


def _main_compute(xs, w_embed, b_embed, wI, wR):
    in_specs = [pl.BlockSpec((1, CHUNK, 16), lambda i: (i, 0, 0)),
                _wspec((ENCF, DDIM)), _wspec((1, DDIM))]
    args = [xs, w_embed, b_embed]
    for wset in (wI, wR):
        for arr in wset:
            in_specs.append(_wspec(arr.shape))
            args.append(arr)
    return pl.pallas_call(
        _main_kernel,
        grid=(NCHUNKS,),
        in_specs=in_specs,
        out_specs=pl.BlockSpec((1, CHUNK, 16), lambda i: (i, 0, 0)),
        out_shape=jax.ShapeDtypeStruct((NCHUNKS, CHUNK, 16), jnp.float32),
    )(*args)


def kernel(X, random_rotations, W_embed, b_embed,
           W_enc_id, b_enc_id, W_t_id, b_t_id, W_h_id, theta_id,
           W_dec_id, b_dec_id, W_out_id, b_out_id,
           W_enc_reg, b_enc_reg, W_t_reg, b_t_reg, W_h_reg, theta_reg,
           W_dec_reg, b_dec_reg, W_out_reg, b_out_reg):
    xp = jnp.pad(X, ((0, 0), (0, 0), (0, 16 - NFEAT)))
    b_embed2 = b_embed.reshape(1, DDIM)

    rank = _compute_rank(xp, W_embed, b_embed2, random_rotations)
    rank_flat = rank.reshape(ROWS)

    tail_id = jnp.arange(ROWS, ROWS_PAD, dtype=jnp.int32)
    dest_idx = jnp.concatenate([rank_flat, tail_id])
    src_idx = jnp.concatenate([rank_flat,
                               jnp.zeros(ROWS_PAD - ROWS, jnp.int32)])

    xp_flat = jnp.concatenate(
        [xp.reshape(ROWS, 16), jnp.zeros((ROWS_PAD - ROWS, 16), jnp.float32)])
    sc_scatter, sc_gather = _sc_kernels()
    xs_flat = sc_scatter(xp_flat, dest_idx)
    xs = xs_flat[:ROWS].reshape(NCHUNKS, CHUNK, 16)

    wI = (W_enc_id, b_enc_id.reshape(1, HID), W_t_id, b_t_id.reshape(1, HID),
          W_h_id, theta_id, W_dec_id, b_dec_id.reshape(1, HID),
          W_out_id, b_out_id.reshape(1, NID))
    wR = (W_enc_reg, b_enc_reg.reshape(1, HID), W_t_reg, b_t_reg.reshape(1, HID),
          W_h_reg, theta_reg, W_dec_reg, b_dec_reg.reshape(1, HID),
          W_out_reg, b_out_reg.reshape(1, NREG))
    y = _main_compute(xs, W_embed, b_embed2, wI, wR)

    y_flat = jnp.concatenate(
        [y.reshape(ROWS, 16), jnp.zeros((ROWS_PAD - ROWS, 16), jnp.float32)])
    out_flat = sc_gather(y_flat, src_idx)
    return out_flat[:ROWS].reshape(B, N, 16)[:, :, :NID + NREG]


# speedup vs baseline: 23.9192x; 1.0674x over previous
import functools

import jax
import jax.numpy as jnp
from jax import lax
from jax.experimental import pallas as pl
from jax.experimental.pallas import tpu as pltpu
from jax.experimental.pallas import tpu_sc as plsc

B = 10
N = 5000
NFEAT = 15
NCLS = 12
ENCF = NCLS + NFEAT - 1
DDIM = 256
HID = 256
NBINS = 10
K = 5
NID = 8
NREG = 4
CHUNK = N // NBINS
NCHUNKS = B * NBINS
ROWS = B * N
SC_CH = 112
NW = 32
ROWS_PAD = ((ROWS + NW * SC_CH - 1) // (NW * SC_CH)) * (NW * SC_CH)
PER_W = ROWS_PAD // NW
NCH = PER_W // SC_CH
RBLK = 1000


def _encode(x):
    m = x.shape[0]
    cls = x[:, 0:1].astype(jnp.int32)
    iota = lax.broadcasted_iota(jnp.int32, (m, NCLS), 1)
    oh = (cls == iota).astype(jnp.float32)
    return jnp.concatenate([oh, x[:, 1:NFEAT]], axis=1)


def _selu(x):
    alpha = 1.6732632423543772848170429916717
    scale = 1.0507009873554804934193349852946
    return scale * jnp.where(x > 0, x, alpha * (jnp.exp(x) - 1.0))


def _rank_kernel(x_ref, wemb_ref, bemb_ref, rot_ref, out_ref):
    b = pl.program_id(0)
    x = x_ref[0]
    enc = _encode(x)
    pts = _selu(jnp.dot(enc, wemb_ref[...], preferred_element_type=jnp.float32)
                + bemb_ref[...])
    mul = jnp.dot(pts, rot_ref[...], preferred_element_type=jnp.float32)
    full = jnp.concatenate([mul, -mul], axis=1)
    mx = jnp.max(full, axis=1, keepdims=True)
    col10 = lax.broadcasted_iota(jnp.int32, (N, NBINS), 1)
    first = jnp.min(jnp.where(full == mx, col10, 127), axis=1, keepdims=True)
    col16 = lax.broadcasted_iota(jnp.int32, (N, 16), 1)
    ohbin = (col16 == first).astype(jnp.float32)

    tot = jnp.sum(ohbin, axis=0, keepdims=True)
    ltmask = (col16 < first).astype(jnp.float32)
    off_node = jnp.sum(ltmask * tot, axis=1, keepdims=True)

    rl = lax.broadcasted_iota(jnp.int32, (RBLK, RBLK), 0)
    cl = lax.broadcasted_iota(jnp.int32, (RBLK, RBLK), 1)
    ltri = (rl > cl).astype(jnp.float32)

    blockoff = jnp.zeros((1, 16), jnp.float32)
    ranks = []
    for blk in range(N // RBLK):
        ohb = ohbin[blk * RBLK:(blk + 1) * RBLK]
        prefix = jnp.dot(ltri, ohb, preferred_element_type=jnp.float32)
        rv = blockoff + prefix
        within = jnp.sum(ohb * rv, axis=1, keepdims=True)
        ranks.append(within + off_node[blk * RBLK:(blk + 1) * RBLK])
        blockoff = blockoff + jnp.sum(ohb, axis=0, keepdims=True)
    rank = jnp.concatenate(ranks, axis=0)
    out_ref[0] = rank.astype(jnp.int32) + b * N


def _compute_rank(xp, w_embed, b_embed, rot):
    return pl.pallas_call(
        _rank_kernel,
        grid=(B,),
        in_specs=[
            pl.BlockSpec((1, N, 16), lambda i: (i, 0, 0)),
            pl.BlockSpec((ENCF, DDIM), lambda i: (0, 0)),
            pl.BlockSpec((1, DDIM), lambda i: (0, 0)),
            pl.BlockSpec((DDIM, NBINS // 2), lambda i: (0, 0)),
        ],
        out_specs=pl.BlockSpec((1, N, 1), lambda i: (i, 0, 0)),
        out_shape=jax.ShapeDtypeStruct((B, N, 1), jnp.int32),
        compiler_params=pltpu.CompilerParams(
            dimension_semantics=("parallel",)),
    )(xp, w_embed, b_embed, rot)


@functools.lru_cache(maxsize=1)
def _sc_kernels():
    mesh = plsc.VectorSubcoreMesh(core_axis_name="c", subcore_axis_name="s")
    scratch = [
        pltpu.VMEM((SC_CH,), jnp.int32),
        pltpu.VMEM((SC_CH, 16), jnp.float32),
        pltpu.SemaphoreType.DMA,
    ]
    out_t = jax.ShapeDtypeStruct((ROWS_PAD, 16), jnp.float32)
    cparams = pltpu.CompilerParams(use_tc_tiling_on_sc=False)

    @functools.partial(pl.kernel, out_type=out_t, mesh=mesh, scratch_types=scratch,
                       compiler_params=cparams)
    def sc_scatter(x_hbm, idx_hbm, out_hbm, idx_v, rows_v, sem):
        wid = lax.axis_index("s") * 2 + lax.axis_index("c")
        base = wid * PER_W
        for j in range(NCH):
            off = base + j * SC_CH
            pltpu.sync_copy(idx_hbm.at[pl.ds(off, SC_CH)], idx_v)
            pltpu.sync_copy(x_hbm.at[pl.ds(off, SC_CH)], rows_v)
            pltpu.async_copy(rows_v, out_hbm.at[idx_v], sem).wait()

    @functools.partial(pl.kernel, out_type=out_t, mesh=mesh, scratch_types=scratch,
                       compiler_params=cparams)
    def sc_gather(y_hbm, idx_hbm, out_hbm, idx_v, rows_v, sem):
        wid = lax.axis_index("s") * 2 + lax.axis_index("c")
        base = wid * PER_W
        for j in range(NCH):
            off = base + j * SC_CH
            pltpu.sync_copy(idx_hbm.at[pl.ds(off, SC_CH)], idx_v)
            pltpu.async_copy(y_hbm.at[idx_v], rows_v, sem).wait()
            pltpu.sync_copy(rows_v, out_hbm.at[pl.ds(off, SC_CH)])

    return sc_scatter, sc_gather


def _main_kernel(xs_ref, wemb_ref, bemb_ref,
                 wencI_ref, bencI_ref, wtI_ref, btI_ref, whI_ref, thI_ref,
                 wdecI_ref, bdecI_ref, woutI_ref, boutI_ref,
                 wencR_ref, bencR_ref, wtR_ref, btR_ref, whR_ref, thR_ref,
                 wdecR_ref, bdecR_ref, woutR_ref, boutR_ref,
                 out_ref):
    xs = xs_ref[0]
    enc = _encode(xs)
    pts = _selu(jnp.dot(enc, wemb_ref[...], preferred_element_type=jnp.float32)
                + bemb_ref[...])
    nrm = pts * lax.rsqrt(jnp.maximum(jnp.sum(pts * pts, axis=1, keepdims=True),
                                      1e-12))
    logits = lax.dot_general(nrm, nrm, (((1,), (1,)), ((), ())),
                             preferred_element_type=jnp.float32)
    lm = jnp.max(logits, axis=1, keepdims=True)
    ex = jnp.exp(logits - lm)
    dm = ex / jnp.sum(ex, axis=1, keepdims=True)

    colf = lax.broadcasted_iota(jnp.int32, (CHUNK, CHUNK), 1).astype(jnp.float32)
    cur = dm
    sel = jnp.zeros((CHUNK, CHUNK), jnp.bool_)
    for _ in range(K):
        m = jnp.max(cur, axis=1, keepdims=True)
        fc = jnp.min(jnp.where(cur == m, colf, 1e9), axis=1, keepdims=True)
        s = colf == fc
        sel = jnp.logical_or(sel, s)
        cur = jnp.where(s, -1.0, cur)
    a_mat = jnp.where(sel, dm, 0.0)
    indeg = jnp.sum(a_mat, axis=1, keepdims=True)
    norm = lax.rsqrt(indeg + 1e-6)

    def branch(wenc, benc, wt, bt, wh, th, wdec, bdec, wout, bout):
        x = _selu(jnp.dot(enc, wenc, preferred_element_type=jnp.float32) + benc)
        fh = jnp.dot(x, th, preferred_element_type=jnp.float32) * norm
        fh = jnp.dot(a_mat, fh, preferred_element_type=jnp.float32) * norm
        gate = jax.nn.sigmoid(jnp.dot(x, wt, preferred_element_type=jnp.float32) + bt)
        het = jnp.dot(x, wh, preferred_element_type=jnp.float32)
        h = _selu(gate * fh + (1.0 - gate) * het)
        h = _selu(jnp.dot(h, wdec, preferred_element_type=jnp.float32) + bdec)
        return jnp.dot(h, wout, preferred_element_type=jnp.float32) + bout

    y_id = branch(wencI_ref[...], bencI_ref[...], wtI_ref[...], btI_ref[...],
                  whI_ref[...], thI_ref[...], wdecI_ref[...], bdecI_ref[...],
                  woutI_ref[...], boutI_ref[...])
    y_rg = branch(wencR_ref[...], bencR_ref[...], wtR_ref[...], btR_ref[...],
                  whR_ref[...], thR_ref[...], wdecR_ref[...], bdecR_ref[...],
                  woutR_ref[...], boutR_ref[...])
    pad = jnp.zeros((CHUNK, 16 - NID - NREG), jnp.float32)
    out_ref[0] = jnp.concatenate([y_id, y_rg, pad], axis=1)


def _wspec(shape):
    nd = len(shape)
    return pl.BlockSpec(shape, lambda i: (0,) * nd)


def _main_compute(xs, w_embed, b_embed, wI, wR):
    in_specs = [pl.BlockSpec((1, CHUNK, 16), lambda i: (i, 0, 0)),
                _wspec((ENCF, DDIM)), _wspec((1, DDIM))]
    args = [xs, w_embed, b_embed]
    for wset in (wI, wR):
        for arr in wset:
            in_specs.append(_wspec(arr.shape))
            args.append(arr)
    return pl.pallas_call(
        _main_kernel,
        grid=(NCHUNKS,),
        in_specs=in_specs,
        out_specs=pl.BlockSpec((1, CHUNK, 16), lambda i: (i, 0, 0)),
        out_shape=jax.ShapeDtypeStruct((NCHUNKS, CHUNK, 16), jnp.float32),
        compiler_params=pltpu.CompilerParams(
            dimension_semantics=("parallel",)),
    )(*args)


def kernel(X, random_rotations, W_embed, b_embed,
           W_enc_id, b_enc_id, W_t_id, b_t_id, W_h_id, theta_id,
           W_dec_id, b_dec_id, W_out_id, b_out_id,
           W_enc_reg, b_enc_reg, W_t_reg, b_t_reg, W_h_reg, theta_reg,
           W_dec_reg, b_dec_reg, W_out_reg, b_out_reg):
    xp = jnp.pad(X, ((0, 0), (0, 0), (0, 16 - NFEAT)))
    b_embed2 = b_embed.reshape(1, DDIM)

    rank = _compute_rank(xp, W_embed, b_embed2, random_rotations)
    rank_flat = rank.reshape(ROWS)

    tail_id = jnp.arange(ROWS, ROWS_PAD, dtype=jnp.int32)
    dest_idx = jnp.concatenate([rank_flat, tail_id])
    src_idx = jnp.concatenate([rank_flat,
                               jnp.zeros(ROWS_PAD - ROWS, jnp.int32)])

    xp_flat = jnp.concatenate(
        [xp.reshape(ROWS, 16), jnp.zeros((ROWS_PAD - ROWS, 16), jnp.float32)])
    sc_scatter, sc_gather = _sc_kernels()
    xs_flat = sc_scatter(xp_flat, dest_idx)
    xs = xs_flat[:ROWS].reshape(NCHUNKS, CHUNK, 16)

    wI = (W_enc_id, b_enc_id.reshape(1, HID), W_t_id, b_t_id.reshape(1, HID),
          W_h_id, theta_id, W_dec_id, b_dec_id.reshape(1, HID),
          W_out_id, b_out_id.reshape(1, NID))
    wR = (W_enc_reg, b_enc_reg.reshape(1, HID), W_t_reg, b_t_reg.reshape(1, HID),
          W_h_reg, theta_reg, W_dec_reg, b_dec_reg.reshape(1, HID),
          W_out_reg, b_out_reg.reshape(1, NREG))
    y = _main_compute(xs, W_embed, b_embed2, wI, wR)

    y_flat = jnp.concatenate(
        [y.reshape(ROWS, 16), jnp.zeros((ROWS_PAD - ROWS, 16), jnp.float32)])
    out_flat = sc_gather(y_flat, src_idx)
    return out_flat[:ROWS].reshape(B, N, 16)[:, :, :NID + NREG]


# speedup vs baseline: 27.0505x; 1.1309x over previous
import functools

import jax
import jax.numpy as jnp
from jax import lax
from jax.experimental import pallas as pl
from jax.experimental.pallas import tpu as pltpu
from jax.experimental.pallas import tpu_sc as plsc

B = 10
N = 5000
NFEAT = 15
NCLS = 12
ENCF = NCLS + NFEAT - 1
DDIM = 256
HID = 256
NBINS = 10
K = 5
NID = 8
NREG = 4
CHUNK = N // NBINS
NCHUNKS = B * NBINS
ROWS = B * N
SC_CH = 112
NW = 32
ROWS_PAD = ((ROWS + NW * SC_CH - 1) // (NW * SC_CH)) * (NW * SC_CH)
PER_W = ROWS_PAD // NW
NCH = PER_W // SC_CH
RBLK = 1000


def _encode(x):
    m = x.shape[0]
    cls = x[:, 0:1].astype(jnp.int32)
    iota = lax.broadcasted_iota(jnp.int32, (m, NCLS), 1)
    oh = (cls == iota).astype(jnp.float32)
    return jnp.concatenate([oh, x[:, 1:NFEAT]], axis=1)


def _selu(x):
    alpha = 1.6732632423543772848170429916717
    scale = 1.0507009873554804934193349852946
    return scale * jnp.where(x > 0, x, alpha * (jnp.exp(x) - 1.0))


def _rank_kernel(x_ref, wemb_ref, bemb_ref, rot_ref, out_ref):
    b = pl.program_id(0)
    x = x_ref[0]
    enc = _encode(x)
    pts = _selu(jnp.dot(enc, wemb_ref[...], preferred_element_type=jnp.float32)
                + bemb_ref[...])
    mul = jnp.dot(pts, rot_ref[...], preferred_element_type=jnp.float32)
    full = jnp.concatenate([mul, -mul], axis=1)
    mx = jnp.max(full, axis=1, keepdims=True)
    col10 = lax.broadcasted_iota(jnp.int32, (N, NBINS), 1)
    first = jnp.min(jnp.where(full == mx, col10, 127), axis=1, keepdims=True)
    col16 = lax.broadcasted_iota(jnp.int32, (N, 16), 1)
    ohbin = (col16 == first).astype(jnp.float32)

    tot = jnp.sum(ohbin, axis=0, keepdims=True)
    ltmask = (col16 < first).astype(jnp.float32)
    off_node = jnp.sum(ltmask * tot, axis=1, keepdims=True)

    rl = lax.broadcasted_iota(jnp.int32, (RBLK, RBLK), 0)
    cl = lax.broadcasted_iota(jnp.int32, (RBLK, RBLK), 1)
    ltri = (rl > cl).astype(jnp.bfloat16)

    blockoff = jnp.zeros((1, 16), jnp.float32)
    ranks = []
    for blk in range(N // RBLK):
        ohb = ohbin[blk * RBLK:(blk + 1) * RBLK]
        prefix = jnp.dot(ltri, ohb.astype(jnp.bfloat16),
                         preferred_element_type=jnp.float32)
        rv = blockoff + prefix
        within = jnp.sum(ohb * rv, axis=1, keepdims=True)
        ranks.append(within + off_node[blk * RBLK:(blk + 1) * RBLK])
        blockoff = blockoff + jnp.sum(ohb, axis=0, keepdims=True)
    rank = jnp.concatenate(ranks, axis=0)
    out_ref[0] = rank.astype(jnp.int32) + b * N


def _compute_rank(xp, w_embed, b_embed, rot):
    return pl.pallas_call(
        _rank_kernel,
        grid=(B,),
        in_specs=[
            pl.BlockSpec((1, N, 16), lambda i: (i, 0, 0)),
            pl.BlockSpec((ENCF, DDIM), lambda i: (0, 0)),
            pl.BlockSpec((1, DDIM), lambda i: (0, 0)),
            pl.BlockSpec((DDIM, NBINS // 2), lambda i: (0, 0)),
        ],
        out_specs=pl.BlockSpec((1, N, 1), lambda i: (i, 0, 0)),
        out_shape=jax.ShapeDtypeStruct((B, N, 1), jnp.int32),
        compiler_params=pltpu.CompilerParams(
            dimension_semantics=("parallel",)),
    )(xp, w_embed, b_embed, rot)


@functools.lru_cache(maxsize=1)
def _sc_kernels():
    mesh = plsc.VectorSubcoreMesh(core_axis_name="c", subcore_axis_name="s")
    scratch = [
        pltpu.VMEM((SC_CH,), jnp.int32),
        pltpu.VMEM((SC_CH, 16), jnp.float32),
        pltpu.SemaphoreType.DMA,
    ]
    out_t = jax.ShapeDtypeStruct((ROWS_PAD, 16), jnp.float32)
    cparams = pltpu.CompilerParams(use_tc_tiling_on_sc=False)

    @functools.partial(pl.kernel, out_type=out_t, mesh=mesh, scratch_types=scratch,
                       compiler_params=cparams)
    def sc_scatter(x_hbm, idx_hbm, out_hbm, idx_v, rows_v, sem):
        wid = lax.axis_index("s") * 2 + lax.axis_index("c")
        base = wid * PER_W
        for j in range(NCH):
            off = base + j * SC_CH
            pltpu.sync_copy(idx_hbm.at[pl.ds(off, SC_CH)], idx_v)
            pltpu.sync_copy(x_hbm.at[pl.ds(off, SC_CH)], rows_v)
            pltpu.async_copy(rows_v, out_hbm.at[idx_v], sem).wait()

    @functools.partial(pl.kernel, out_type=out_t, mesh=mesh, scratch_types=scratch,
                       compiler_params=cparams)
    def sc_gather(y_hbm, idx_hbm, out_hbm, idx_v, rows_v, sem):
        wid = lax.axis_index("s") * 2 + lax.axis_index("c")
        base = wid * PER_W
        for j in range(NCH):
            off = base + j * SC_CH
            pltpu.sync_copy(idx_hbm.at[pl.ds(off, SC_CH)], idx_v)
            pltpu.async_copy(y_hbm.at[idx_v], rows_v, sem).wait()
            pltpu.sync_copy(rows_v, out_hbm.at[pl.ds(off, SC_CH)])

    return sc_scatter, sc_gather


def _main_kernel(xs_ref, wemb_ref, bemb_ref,
                 wencI_ref, bencI_ref, wtI_ref, btI_ref, whI_ref, thI_ref,
                 wdecI_ref, bdecI_ref, woutI_ref, boutI_ref,
                 wencR_ref, bencR_ref, wtR_ref, btR_ref, whR_ref, thR_ref,
                 wdecR_ref, bdecR_ref, woutR_ref, boutR_ref,
                 out_ref):
    xs = xs_ref[0]
    enc = _encode(xs)
    pts = _selu(jnp.dot(enc, wemb_ref[...], preferred_element_type=jnp.float32)
                + bemb_ref[...])
    nrm = pts * lax.rsqrt(jnp.maximum(jnp.sum(pts * pts, axis=1, keepdims=True),
                                      1e-12))
    logits = lax.dot_general(nrm, nrm, (((1,), (1,)), ((), ())),
                             preferred_element_type=jnp.float32)
    ex = jnp.exp(logits)
    rec_s = 1.0 / jnp.sum(ex, axis=1, keepdims=True)

    v = jnp.max(ex, axis=1, keepdims=True)
    for _ in range(K - 1):
        v = jnp.max(jnp.where(ex < v, ex, -1.0), axis=1, keepdims=True)
    e_sel = jnp.where(ex >= v, ex, 0.0)
    indeg = jnp.sum(e_sel, axis=1, keepdims=True) * rec_s
    norm = lax.rsqrt(indeg + 1e-6)
    agg_scale = norm * rec_s

    a_bf = e_sel.astype(jnp.bfloat16)

    def branch(wenc, benc, wt, bt, wh, th, wdec, bdec, wout, bout):
        x = _selu(jnp.dot(enc, wenc, preferred_element_type=jnp.float32) + benc)
        xb = x.astype(jnp.bfloat16)
        fh = jnp.dot(xb, th, preferred_element_type=jnp.float32) * norm
        fh = jnp.dot(a_bf, fh.astype(jnp.bfloat16),
                     preferred_element_type=jnp.float32) * agg_scale
        gate = jax.nn.sigmoid(jnp.dot(xb, wt, preferred_element_type=jnp.float32) + bt)
        het = jnp.dot(xb, wh, preferred_element_type=jnp.float32)
        h = _selu(gate * fh + (1.0 - gate) * het)
        hb = h.astype(jnp.bfloat16)
        h2 = _selu(jnp.dot(hb, wdec, preferred_element_type=jnp.float32) + bdec)
        return jnp.dot(h2.astype(jnp.bfloat16), wout,
                       preferred_element_type=jnp.float32) + bout

    y_id = branch(wencI_ref[...], bencI_ref[...], wtI_ref[...], btI_ref[...],
                  whI_ref[...], thI_ref[...], wdecI_ref[...], bdecI_ref[...],
                  woutI_ref[...], boutI_ref[...])
    y_rg = branch(wencR_ref[...], bencR_ref[...], wtR_ref[...], btR_ref[...],
                  whR_ref[...], thR_ref[...], wdecR_ref[...], bdecR_ref[...],
                  woutR_ref[...], boutR_ref[...])
    pad = jnp.zeros((CHUNK, 16 - NID - NREG), jnp.float32)
    out_ref[0] = jnp.concatenate([y_id, y_rg, pad], axis=1)


def _wspec(shape):
    nd = len(shape)
    return pl.BlockSpec(shape, lambda i: (0,) * nd)


def _main_compute(xs, w_embed, b_embed, wI, wR):
    in_specs = [pl.BlockSpec((1, CHUNK, 16), lambda i: (i, 0, 0)),
                _wspec((ENCF, DDIM)), _wspec((1, DDIM))]
    args = [xs, w_embed, b_embed]
    for wset in (wI, wR):
        for arr in wset:
            in_specs.append(_wspec(arr.shape))
            args.append(arr)
    return pl.pallas_call(
        _main_kernel,
        grid=(NCHUNKS,),
        in_specs=in_specs,
        out_specs=pl.BlockSpec((1, CHUNK, 16), lambda i: (i, 0, 0)),
        out_shape=jax.ShapeDtypeStruct((NCHUNKS, CHUNK, 16), jnp.float32),
        compiler_params=pltpu.CompilerParams(
            dimension_semantics=("parallel",)),
    )(*args)


def kernel(X, random_rotations, W_embed, b_embed,
           W_enc_id, b_enc_id, W_t_id, b_t_id, W_h_id, theta_id,
           W_dec_id, b_dec_id, W_out_id, b_out_id,
           W_enc_reg, b_enc_reg, W_t_reg, b_t_reg, W_h_reg, theta_reg,
           W_dec_reg, b_dec_reg, W_out_reg, b_out_reg):
    xp = jnp.pad(X, ((0, 0), (0, 0), (0, 16 - NFEAT)))
    b_embed2 = b_embed.reshape(1, DDIM)

    rank = _compute_rank(xp, W_embed, b_embed2, random_rotations)
    rank_flat = rank.reshape(ROWS)

    tail_id = jnp.arange(ROWS, ROWS_PAD, dtype=jnp.int32)
    dest_idx = jnp.concatenate([rank_flat, tail_id])
    src_idx = jnp.concatenate([rank_flat,
                               jnp.zeros(ROWS_PAD - ROWS, jnp.int32)])

    xp_flat = jnp.concatenate(
        [xp.reshape(ROWS, 16), jnp.zeros((ROWS_PAD - ROWS, 16), jnp.float32)])
    sc_scatter, sc_gather = _sc_kernels()
    xs_flat = sc_scatter(xp_flat, dest_idx)
    xs = xs_flat[:ROWS].reshape(NCHUNKS, CHUNK, 16)

    bf = jnp.bfloat16
    wI = (W_enc_id, b_enc_id.reshape(1, HID), W_t_id.astype(bf),
          b_t_id.reshape(1, HID), W_h_id.astype(bf), theta_id.astype(bf),
          W_dec_id.astype(bf), b_dec_id.reshape(1, HID),
          W_out_id.astype(bf), b_out_id.reshape(1, NID))
    wR = (W_enc_reg, b_enc_reg.reshape(1, HID), W_t_reg.astype(bf),
          b_t_reg.reshape(1, HID), W_h_reg.astype(bf), theta_reg.astype(bf),
          W_dec_reg.astype(bf), b_dec_reg.reshape(1, HID),
          W_out_reg.astype(bf), b_out_reg.reshape(1, NREG))
    y = _main_compute(xs, W_embed, b_embed2, wI, wR)

    y_flat = jnp.concatenate(
        [y.reshape(ROWS, 16), jnp.zeros((ROWS_PAD - ROWS, 16), jnp.float32)])
    out_flat = sc_gather(y_flat, src_idx)
    return out_flat[:ROWS].reshape(B, N, 16)[:, :, :NID + NREG]


# speedup vs baseline: 28.1680x; 1.0413x over previous
import functools

import jax
import jax.numpy as jnp
from jax import lax
from jax.experimental import pallas as pl
from jax.experimental.pallas import tpu as pltpu
from jax.experimental.pallas import tpu_sc as plsc

B = 10
N = 5000
NFEAT = 15
NCLS = 12
ENCF = NCLS + NFEAT - 1
DDIM = 256
HID = 256
NBINS = 10
K = 5
NID = 8
NREG = 4
CHUNK = N // NBINS
NCHUNKS = B * NBINS
ROWS = B * N
SC_CH = 112
NW = 32
ROWS_PAD = ((ROWS + NW * SC_CH - 1) // (NW * SC_CH)) * (NW * SC_CH)
PER_W = ROWS_PAD // NW
NCH = PER_W // SC_CH
RBLK = 1000


def _encode(x):
    m = x.shape[0]
    cls = x[:, 0:1].astype(jnp.int32)
    iota = lax.broadcasted_iota(jnp.int32, (m, NCLS), 1)
    oh = (cls == iota).astype(jnp.float32)
    return jnp.concatenate([oh, x[:, 1:NFEAT]], axis=1)


def _selu(x):
    alpha = 1.6732632423543772848170429916717
    scale = 1.0507009873554804934193349852946
    return scale * jnp.where(x > 0, x, alpha * (jnp.exp(x) - 1.0))


def _rank_kernel(x_ref, wemb_ref, bemb_ref, rot_ref, out_ref):
    b = pl.program_id(0)
    x = x_ref[0]
    enc = _encode(x)
    pts = _selu(jnp.dot(enc, wemb_ref[...], preferred_element_type=jnp.float32)
                + bemb_ref[...])
    mul = jnp.dot(pts, rot_ref[...], preferred_element_type=jnp.float32)
    full = jnp.concatenate([mul, -mul], axis=1)
    mx = jnp.max(full, axis=1, keepdims=True)
    col10 = lax.broadcasted_iota(jnp.int32, (N, NBINS), 1)
    first = jnp.min(jnp.where(full == mx, col10, 127), axis=1, keepdims=True)
    col16 = lax.broadcasted_iota(jnp.int32, (N, 16), 1)
    ohbin = (col16 == first).astype(jnp.float32)

    tot = jnp.sum(ohbin, axis=0, keepdims=True)
    ltmask = (col16 < first).astype(jnp.float32)
    off_node = jnp.sum(ltmask * tot, axis=1, keepdims=True)

    rl = lax.broadcasted_iota(jnp.int32, (RBLK, RBLK), 0)
    cl = lax.broadcasted_iota(jnp.int32, (RBLK, RBLK), 1)
    ltri = (rl > cl).astype(jnp.bfloat16)

    blockoff = jnp.zeros((1, 16), jnp.float32)
    ranks = []
    for blk in range(N // RBLK):
        ohb = ohbin[blk * RBLK:(blk + 1) * RBLK]
        prefix = jnp.dot(ltri, ohb.astype(jnp.bfloat16),
                         preferred_element_type=jnp.float32)
        rv = blockoff + prefix
        within = jnp.sum(ohb * rv, axis=1, keepdims=True)
        ranks.append(within + off_node[blk * RBLK:(blk + 1) * RBLK])
        blockoff = blockoff + jnp.sum(ohb, axis=0, keepdims=True)
    rank = jnp.concatenate(ranks, axis=0)
    out_ref[0] = rank.astype(jnp.int32) + b * N


def _compute_rank(xp, w_embed, b_embed, rot):
    return pl.pallas_call(
        _rank_kernel,
        grid=(B,),
        in_specs=[
            pl.BlockSpec((1, N, 16), lambda i: (i, 0, 0)),
            pl.BlockSpec((ENCF, DDIM), lambda i: (0, 0)),
            pl.BlockSpec((1, DDIM), lambda i: (0, 0)),
            pl.BlockSpec((DDIM, NBINS // 2), lambda i: (0, 0)),
        ],
        out_specs=pl.BlockSpec((1, N, 1), lambda i: (i, 0, 0)),
        out_shape=jax.ShapeDtypeStruct((B, N, 1), jnp.int32),
        compiler_params=pltpu.CompilerParams(
            dimension_semantics=("parallel",)),
    )(xp, w_embed, b_embed, rot)


@functools.lru_cache(maxsize=1)
def _sc_kernels():
    mesh = plsc.VectorSubcoreMesh(core_axis_name="c", subcore_axis_name="s")
    scratch = [
        pltpu.VMEM((NCH, SC_CH), jnp.int32),
        pltpu.VMEM((PER_W, 16), jnp.float32),
        pltpu.SemaphoreType.DMA,
    ]
    out_t = jax.ShapeDtypeStruct((ROWS_PAD, 16), jnp.float32)
    cparams = pltpu.CompilerParams(use_tc_tiling_on_sc=False)

    @functools.partial(pl.kernel, out_type=out_t, mesh=mesh, scratch_types=scratch,
                       compiler_params=cparams)
    def sc_scatter(x_hbm, idx_hbm, out_hbm, idx_v, rows_v, sem):
        wid = lax.axis_index("s") * 2 + lax.axis_index("c")
        base = wid * PER_W
        pltpu.sync_copy(idx_hbm.at[wid], idx_v)
        pltpu.sync_copy(x_hbm.at[pl.ds(base, PER_W)], rows_v)
        descs = [
            pltpu.async_copy(rows_v.at[pl.ds(j * SC_CH, SC_CH)],
                             out_hbm.at[idx_v.at[j]], sem)
            for j in range(NCH)
        ]
        for d in descs:
            d.wait()

    @functools.partial(pl.kernel, out_type=out_t, mesh=mesh, scratch_types=scratch,
                       compiler_params=cparams)
    def sc_gather(y_hbm, idx_hbm, out_hbm, idx_v, rows_v, sem):
        wid = lax.axis_index("s") * 2 + lax.axis_index("c")
        base = wid * PER_W
        pltpu.sync_copy(idx_hbm.at[wid], idx_v)
        descs = [
            pltpu.async_copy(y_hbm.at[idx_v.at[j]],
                             rows_v.at[pl.ds(j * SC_CH, SC_CH)], sem)
            for j in range(NCH)
        ]
        for d in descs:
            d.wait()
        pltpu.sync_copy(rows_v, out_hbm.at[pl.ds(base, PER_W)])

    return sc_scatter, sc_gather


def _main_kernel(xs_ref, wemb_ref, bemb_ref,
                 wencI_ref, bencI_ref, wtI_ref, btI_ref, whI_ref, thI_ref,
                 wdecI_ref, bdecI_ref, woutI_ref, boutI_ref,
                 wencR_ref, bencR_ref, wtR_ref, btR_ref, whR_ref, thR_ref,
                 wdecR_ref, bdecR_ref, woutR_ref, boutR_ref,
                 out_ref):
    xs = xs_ref[0]
    enc = _encode(xs)
    pts = _selu(jnp.dot(enc, wemb_ref[...], preferred_element_type=jnp.float32)
                + bemb_ref[...])
    nrm = pts * lax.rsqrt(jnp.maximum(jnp.sum(pts * pts, axis=1, keepdims=True),
                                      1e-12))
    logits = lax.dot_general(nrm, nrm, (((1,), (1,)), ((), ())),
                             preferred_element_type=jnp.float32)
    ex = jnp.exp(logits)
    rec_s = 1.0 / jnp.sum(ex, axis=1, keepdims=True)

    v = jnp.max(ex, axis=1, keepdims=True)
    for _ in range(K - 1):
        v = jnp.max(jnp.where(ex < v, ex, -1.0), axis=1, keepdims=True)
    e_sel = jnp.where(ex >= v, ex, 0.0)
    indeg = jnp.sum(e_sel, axis=1, keepdims=True) * rec_s
    norm = lax.rsqrt(indeg + 1e-6)
    agg_scale = norm * rec_s

    a_bf = e_sel.astype(jnp.bfloat16)

    def branch(wenc, benc, wt, bt, wh, th, wdec, bdec, wout, bout):
        x = _selu(jnp.dot(enc, wenc, preferred_element_type=jnp.float32) + benc)
        xb = x.astype(jnp.bfloat16)
        fh = jnp.dot(xb, th, preferred_element_type=jnp.float32) * norm
        fh = jnp.dot(a_bf, fh.astype(jnp.bfloat16),
                     preferred_element_type=jnp.float32) * agg_scale
        gate = jax.nn.sigmoid(jnp.dot(xb, wt, preferred_element_type=jnp.float32) + bt)
        het = jnp.dot(xb, wh, preferred_element_type=jnp.float32)
        h = _selu(gate * fh + (1.0 - gate) * het)
        hb = h.astype(jnp.bfloat16)
        h2 = _selu(jnp.dot(hb, wdec, preferred_element_type=jnp.float32) + bdec)
        return jnp.dot(h2.astype(jnp.bfloat16), wout,
                       preferred_element_type=jnp.float32) + bout

    y_id = branch(wencI_ref[...], bencI_ref[...], wtI_ref[...], btI_ref[...],
                  whI_ref[...], thI_ref[...], wdecI_ref[...], bdecI_ref[...],
                  woutI_ref[...], boutI_ref[...])
    y_rg = branch(wencR_ref[...], bencR_ref[...], wtR_ref[...], btR_ref[...],
                  whR_ref[...], thR_ref[...], wdecR_ref[...], bdecR_ref[...],
                  woutR_ref[...], boutR_ref[...])
    pad = jnp.zeros((CHUNK, 16 - NID - NREG), jnp.float32)
    out_ref[0] = jnp.concatenate([y_id, y_rg, pad], axis=1)


def _wspec(shape):
    nd = len(shape)
    return pl.BlockSpec(shape, lambda i: (0,) * nd)


def _main_compute(xs, w_embed, b_embed, wI, wR):
    in_specs = [pl.BlockSpec((1, CHUNK, 16), lambda i: (i, 0, 0)),
                _wspec((ENCF, DDIM)), _wspec((1, DDIM))]
    args = [xs, w_embed, b_embed]
    for wset in (wI, wR):
        for arr in wset:
            in_specs.append(_wspec(arr.shape))
            args.append(arr)
    return pl.pallas_call(
        _main_kernel,
        grid=(NCHUNKS,),
        in_specs=in_specs,
        out_specs=pl.BlockSpec((1, CHUNK, 16), lambda i: (i, 0, 0)),
        out_shape=jax.ShapeDtypeStruct((NCHUNKS, CHUNK, 16), jnp.float32),
        compiler_params=pltpu.CompilerParams(
            dimension_semantics=("parallel",)),
    )(*args)


def kernel(X, random_rotations, W_embed, b_embed,
           W_enc_id, b_enc_id, W_t_id, b_t_id, W_h_id, theta_id,
           W_dec_id, b_dec_id, W_out_id, b_out_id,
           W_enc_reg, b_enc_reg, W_t_reg, b_t_reg, W_h_reg, theta_reg,
           W_dec_reg, b_dec_reg, W_out_reg, b_out_reg):
    xp = jnp.pad(X, ((0, 0), (0, 0), (0, 16 - NFEAT)))
    b_embed2 = b_embed.reshape(1, DDIM)

    rank = _compute_rank(xp, W_embed, b_embed2, random_rotations)
    rank_flat = rank.reshape(ROWS)

    tail_id = jnp.arange(ROWS, ROWS_PAD, dtype=jnp.int32)
    dest_idx = jnp.concatenate([rank_flat, tail_id])
    src_idx = jnp.concatenate([rank_flat,
                               jnp.zeros(ROWS_PAD - ROWS, jnp.int32)])

    xp_flat = jnp.concatenate(
        [xp.reshape(ROWS, 16), jnp.zeros((ROWS_PAD - ROWS, 16), jnp.float32)])
    sc_scatter, sc_gather = _sc_kernels()
    dest_idx = dest_idx.reshape(NW, NCH, SC_CH)
    src_idx = src_idx.reshape(NW, NCH, SC_CH)
    xs_flat = sc_scatter(xp_flat, dest_idx)
    xs = xs_flat[:ROWS].reshape(NCHUNKS, CHUNK, 16)

    bf = jnp.bfloat16
    wI = (W_enc_id, b_enc_id.reshape(1, HID), W_t_id.astype(bf),
          b_t_id.reshape(1, HID), W_h_id.astype(bf), theta_id.astype(bf),
          W_dec_id.astype(bf), b_dec_id.reshape(1, HID),
          W_out_id.astype(bf), b_out_id.reshape(1, NID))
    wR = (W_enc_reg, b_enc_reg.reshape(1, HID), W_t_reg.astype(bf),
          b_t_reg.reshape(1, HID), W_h_reg.astype(bf), theta_reg.astype(bf),
          W_dec_reg.astype(bf), b_dec_reg.reshape(1, HID),
          W_out_reg.astype(bf), b_out_reg.reshape(1, NREG))
    y = _main_compute(xs, W_embed, b_embed2, wI, wR)

    y_flat = jnp.concatenate(
        [y.reshape(ROWS, 16), jnp.zeros((ROWS_PAD - ROWS, 16), jnp.float32)])
    out_flat = sc_gather(y_flat, src_idx)
    return out_flat[:ROWS].reshape(B, N, 16)[:, :, :NID + NREG]
